# baseline (speedup 1.0000x reference)
import jax
import jax.numpy as jnp
from jax.experimental import pallas as pl
from jax.experimental.pallas import tpu as pltpu

B, N, D = 128, 256, 128
DE, DA = 4, 32
H1, H2 = 128, 64


def _encoder_body(er_ref, adj_ref, nodes_ref, m_ref, be_ref, w1a_ref, w1b_ref,
                  b1_ref, w2a_ref, w2b_ref, b2_ref, out_ref):
    er = er_ref[0]
    adj = adj_ref[0]
    h0 = nodes_ref[0]
    g2 = jnp.dot(er, m_ref[...], preferred_element_type=jnp.float32)
    gate = jax.nn.sigmoid(g2 + be_ref[0, 0])
    a = adj * gate
    m1 = jnp.dot(a, h0, preferred_element_type=jnp.float32)
    h1 = jnp.dot(h0, w1a_ref[...], preferred_element_type=jnp.float32)
    h1 += jnp.dot(m1, w1b_ref[...], preferred_element_type=jnp.float32)
    h1 = jax.nn.relu(h1 + b1_ref[...])
    m2 = jnp.dot(a, h1, preferred_element_type=jnp.float32)
    h2 = jnp.dot(h1, w2a_ref[...], preferred_element_type=jnp.float32)
    h2 += jnp.dot(m2, w2b_ref[...], preferred_element_type=jnp.float32)
    h2 = jax.nn.relu(h2 + b2_ref[...])
    out_ref[0, :, :] = jnp.sum(h2, axis=0, keepdims=True) * (1.0 / N)


def _critic_body(g_ref, act_ref, wo_ref, bo_ref, wc1a_ref, wc1b_ref, bc1_ref,
                 wc2_ref, bc2_ref, wv_ref, bv_ref, out_ref):
    g = g_ref[...]
    emb = jnp.dot(g, wo_ref[...], preferred_element_type=jnp.float32) + bo_ref[...]
    x = jnp.dot(emb, wc1a_ref[...], preferred_element_type=jnp.float32)
    x += jnp.dot(act_ref[...], wc1b_ref[...], preferred_element_type=jnp.float32)
    x = jax.nn.relu(x + bc1_ref[...])
    x = jax.nn.relu(jnp.dot(x, wc2_ref[...], preferred_element_type=jnp.float32)
                    + bc2_ref[...])
    out_ref[...] = jnp.dot(x, wv_ref[...], preferred_element_type=jnp.float32) \
        + bv_ref[...]


@jax.jit
def kernel(nodes, edges, adjacency, actions, We, be, W1, b1, W2, b2, Wo, bo,
           Wc1, bc1, Wc2, bc2, Wv, bv):
    er = edges.reshape(B, N, N * DE)
    rows = jnp.arange(N * DE)
    cols = jnp.arange(N)
    m = jnp.where((rows[:, None] // DE) == cols[None, :],
                  We[rows % DE, 0][:, None], 0.0)
    m = m.astype(jnp.float32)

    w1a, w1b = W1[:D], W1[D:]
    w2a, w2b = W2[:H1], W2[H1:]

    gmean = pl.pallas_call(
        _encoder_body,
        grid=(B,),
        in_specs=[
            pl.BlockSpec((1, N, N * DE), lambda b: (b, 0, 0)),
            pl.BlockSpec((1, N, N), lambda b: (b, 0, 0)),
            pl.BlockSpec((1, N, D), lambda b: (b, 0, 0)),
            pl.BlockSpec((N * DE, N), lambda b: (0, 0)),
            pl.BlockSpec((1, 1), lambda b: (0, 0)),
            pl.BlockSpec((D, H1), lambda b: (0, 0)),
            pl.BlockSpec((D, H1), lambda b: (0, 0)),
            pl.BlockSpec((1, H1), lambda b: (0, 0)),
            pl.BlockSpec((H1, H2), lambda b: (0, 0)),
            pl.BlockSpec((H1, H2), lambda b: (0, 0)),
            pl.BlockSpec((1, H2), lambda b: (0, 0)),
        ],
        out_specs=pl.BlockSpec((1, 1, H2), lambda b: (b, 0, 0)),
        out_shape=jax.ShapeDtypeStruct((B, 1, H2), jnp.float32),
        compiler_params=pltpu.CompilerParams(
            dimension_semantics=("arbitrary",),
        ),
    )(er, adjacency, nodes, m, be.reshape(1, 1), w1a, w1b, b1.reshape(1, H1),
      w2a, w2b, b2.reshape(1, H2))

    gmean = gmean.reshape(B, H2)
    wc1a, wc1b = Wc1[:H2], Wc1[H2:]

    q = pl.pallas_call(
        _critic_body,
        out_shape=jax.ShapeDtypeStruct((B, 1), jnp.float32),
    )(gmean, actions, Wo, bo.reshape(1, H2), wc1a, wc1b, bc1.reshape(1, H1),
      Wc2, bc2.reshape(1, H2), Wv, bv.reshape(1, 1))

    return q.reshape(B)


# speedup vs baseline: 1.0002x; 1.0002x over previous
import jax
import jax.numpy as jnp
from jax.experimental import pallas as pl
from jax.experimental.pallas import tpu as pltpu

B, N, D = 128, 256, 128
DE, DA = 4, 32
H1, H2 = 128, 64


def _encoder_body(er_ref, adj_ref, nodes_ref, m_ref, be_ref, w1a_ref, w1b_ref,
                  b1_ref, w2a_ref, w2b_ref, b2_ref, out_ref):
    er = er_ref[0]
    adj = adj_ref[0]
    h0 = nodes_ref[0]
    g2 = jnp.dot(er.astype(jnp.bfloat16), m_ref[...].astype(jnp.bfloat16),
                 preferred_element_type=jnp.float32)
    gate = jax.nn.sigmoid(g2 + be_ref[0, 0])
    a = adj * gate
    m1 = jnp.dot(a, h0, preferred_element_type=jnp.float32)
    h1 = jnp.dot(h0, w1a_ref[...], preferred_element_type=jnp.float32)
    h1 += jnp.dot(m1, w1b_ref[...], preferred_element_type=jnp.float32)
    h1 = jax.nn.relu(h1 + b1_ref[...])
    m2 = jnp.dot(a, h1, preferred_element_type=jnp.float32)
    h2 = jnp.dot(h1, w2a_ref[...], preferred_element_type=jnp.float32)
    h2 += jnp.dot(m2, w2b_ref[...], preferred_element_type=jnp.float32)
    h2 = jax.nn.relu(h2 + b2_ref[...])
    out_ref[0, :, :] = jnp.sum(h2, axis=0, keepdims=True) * (1.0 / N)


def _critic_body(g_ref, act_ref, wo_ref, bo_ref, wc1a_ref, wc1b_ref, bc1_ref,
                 wc2_ref, bc2_ref, wv_ref, bv_ref, out_ref):
    g = g_ref[...]
    emb = jnp.dot(g, wo_ref[...], preferred_element_type=jnp.float32) + bo_ref[...]
    x = jnp.dot(emb, wc1a_ref[...], preferred_element_type=jnp.float32)
    x += jnp.dot(act_ref[...], wc1b_ref[...], preferred_element_type=jnp.float32)
    x = jax.nn.relu(x + bc1_ref[...])
    x = jax.nn.relu(jnp.dot(x, wc2_ref[...], preferred_element_type=jnp.float32)
                    + bc2_ref[...])
    out_ref[...] = jnp.dot(x, wv_ref[...], preferred_element_type=jnp.float32) \
        + bv_ref[...]


@jax.jit
def kernel(nodes, edges, adjacency, actions, We, be, W1, b1, W2, b2, Wo, bo,
           Wc1, bc1, Wc2, bc2, Wv, bv):
    er = edges.reshape(B, N, N * DE)
    rows = jnp.arange(N * DE)
    cols = jnp.arange(N)
    m = jnp.where((rows[:, None] // DE) == cols[None, :],
                  We[rows % DE, 0][:, None], 0.0)
    m = m.astype(jnp.float32)

    w1a, w1b = W1[:D], W1[D:]
    w2a, w2b = W2[:H1], W2[H1:]

    gmean = pl.pallas_call(
        _encoder_body,
        grid=(B,),
        in_specs=[
            pl.BlockSpec((1, N, N * DE), lambda b: (b, 0, 0)),
            pl.BlockSpec((1, N, N), lambda b: (b, 0, 0)),
            pl.BlockSpec((1, N, D), lambda b: (b, 0, 0)),
            pl.BlockSpec((N * DE, N), lambda b: (0, 0)),
            pl.BlockSpec((1, 1), lambda b: (0, 0)),
            pl.BlockSpec((D, H1), lambda b: (0, 0)),
            pl.BlockSpec((D, H1), lambda b: (0, 0)),
            pl.BlockSpec((1, H1), lambda b: (0, 0)),
            pl.BlockSpec((H1, H2), lambda b: (0, 0)),
            pl.BlockSpec((H1, H2), lambda b: (0, 0)),
            pl.BlockSpec((1, H2), lambda b: (0, 0)),
        ],
        out_specs=pl.BlockSpec((1, 1, H2), lambda b: (b, 0, 0)),
        out_shape=jax.ShapeDtypeStruct((B, 1, H2), jnp.float32),
        compiler_params=pltpu.CompilerParams(
            dimension_semantics=("parallel",),
        ),
    )(er, adjacency, nodes, m, be.reshape(1, 1), w1a, w1b, b1.reshape(1, H1),
      w2a, w2b, b2.reshape(1, H2))

    gmean = gmean.reshape(B, H2)
    wc1a, wc1b = Wc1[:H2], Wc1[H2:]

    q = pl.pallas_call(
        _critic_body,
        out_shape=jax.ShapeDtypeStruct((B, 1), jnp.float32),
    )(gmean, actions, Wo, bo.reshape(1, H2), wc1a, wc1b, bc1.reshape(1, H1),
      Wc2, bc2.reshape(1, H2), Wv, bv.reshape(1, 1))

    return q.reshape(B)
